# baseline (speedup 1.0000x reference)
import dataclasses
import functools

import jax
import jax.numpy as jnp
from jax import lax
from jax.experimental import pallas as pl
from jax.experimental.pallas import tpu as pltpu
from jax.experimental.pallas import tpu_sc as plsc

N = 10000
E = 320000
D = 128
NC, NS = 2, 16
C = 128
NP = 10240
RPT = NP // NS
K = 80
ET = K * C
EC = ET * NS
E_PAD = EC * NC
BM = 256

_mesh = plsc.VectorSubcoreMesh(
    core_axis_name="c", subcore_axis_name="s", num_cores=NC, num_subcores=NS
)

_cp = dataclasses.replace(pltpu.CompilerParams(), needs_layout_passes=False)


@functools.partial(
    pl.kernel,
    out_type=jax.ShapeDtypeStruct((NC * NS, NP), jnp.float32),
    mesh=_mesh,
    compiler_params=_cp,
    scratch_types=[
        pltpu.VMEM((K, C), jnp.int32),
        pltpu.VMEM((NP,), jnp.float32),
    ],
)
def _sc_degree(dstr_hbm, zeros_hbm, out_hbm, didx, cnt_v):
    c = lax.axis_index("c")
    sid = lax.axis_index("s")
    pltpu.sync_copy(zeros_hbm, cnt_v)
    pltpu.sync_copy(dstr_hbm.at[c, sid], didx)
    ones = jnp.full((16,), 1.0, jnp.float32)

    @pl.loop(0, K)
    def _(k):
        row = didx.at[k]

        @pl.loop(0, C // 16)
        def _(j):
            idx = row[pl.ds(j * 16, 16)]
            plsc.addupdate_scatter(cnt_v, [idx], ones)

    pltpu.sync_copy(cnt_v, out_hbm.at[c * NS + sid])


@functools.partial(
    pl.kernel,
    out_type=jax.ShapeDtypeStruct((NC, NP, D), jnp.float32),
    mesh=_mesh,
    scratch_types=[
        pltpu.VMEM((K, C), jnp.int32),
        pltpu.VMEM((C,), jnp.int32),
        pltpu.VMEM((C,), jnp.int32),
        pltpu.VMEM((C, D), jnp.float32),
        pltpu.VMEM((C, D), jnp.float32),
        pltpu.VMEM_SHARED((NP, D), jnp.float32),
        pltpu.SemaphoreType.DMA,
        pltpu.SemaphoreType.DMA,
        pltpu.SemaphoreType.DMA,
        pltpu.SemaphoreType.DMA,
        pltpu.SemaphoreType.DMA,
        pltpu.SemaphoreType.DMA,
    ],
)
def _sc_aggregate(h_hbm, srcr_hbm, dstr_hbm, out_hbm, didx, si0, si1,
                  r0, r1, acc_sh, i0, i1, g0, g1, s0, s1):
    c = lax.axis_index("c")
    sid = lax.axis_index("s")
    rows = pl.ds(sid * RPT, RPT)
    srct = srcr_hbm.at[c, sid]
    pltpu.sync_copy(dstr_hbm.at[c, sid], didx)
    pltpu.sync_copy(h_hbm.at[rows], acc_sh.at[rows])
    plsc.subcore_barrier()
    sbufs = (si0, si1)
    bufs = (r0, r1)
    isems = (i0, i1)
    gsems = (g0, g1)
    ssems = (s0, s1)
    pltpu.sync_copy(srct.at[0], si0)
    pltpu.async_copy(srct.at[1], si1, i1)
    pltpu.async_copy(h_hbm.at[si0], r0, g0)

    @pl.loop(0, K, step=2)
    def _(k):
        for b in range(2):
            kk = k + b
            b2 = (b + 1) % 2
            pltpu.make_async_copy(h_hbm.at[sbufs[b]], bufs[b], gsems[b]).wait()

            @pl.when(kk + 2 < K)
            def _():
                pltpu.async_copy(srct.at[kk + 2], sbufs[b], isems[b])

            pltpu.async_copy(bufs[b], acc_sh.at[didx.at[kk]], ssems[b], add=True)

            @pl.when(kk >= 1)
            def _():
                pltpu.make_async_copy(
                    bufs[b2], acc_sh.at[didx.at[kk - 1]], ssems[b2]
                ).wait()

            @pl.when(kk + 1 < K)
            def _():
                pltpu.make_async_copy(srct.at[kk + 1], sbufs[b2], isems[b2]).wait()
                pltpu.async_copy(h_hbm.at[sbufs[b2]], bufs[b2], gsems[b2])

    pltpu.make_async_copy(
        bufs[(K - 1) % 2], acc_sh.at[didx.at[K - 1]], ssems[(K - 1) % 2]
    ).wait()
    plsc.subcore_barrier()
    pltpu.sync_copy(acc_sh.at[rows], out_hbm.at[c, rows])


def _mm(x, W):
    def body(x_ref, w_ref, o_ref):
        o_ref[...] = jax.lax.dot(
            x_ref[...], w_ref[...], precision=jax.lax.Precision.HIGHEST
        )

    return pl.pallas_call(
        body,
        grid=(NP // BM,),
        in_specs=[
            pl.BlockSpec((BM, D), lambda i: (i, 0)),
            pl.BlockSpec((D, D), lambda i: (0, 0)),
        ],
        out_specs=pl.BlockSpec((BM, D), lambda i: (i, 0)),
        out_shape=jax.ShapeDtypeStruct((NP, D), jnp.float32),
    )(x, W)


def _scale(h, degt):
    def body(h_ref, g_ref, hp_ref, dis_ref):
        deg = jnp.sum(g_ref[...], axis=1, keepdims=True) + 1.0
        dis = jax.lax.rsqrt(deg)
        hp_ref[...] = h_ref[...] * dis
        dis_ref[...] = dis

    return pl.pallas_call(
        body,
        grid=(NP // BM,),
        in_specs=[
            pl.BlockSpec((BM, D), lambda i: (i, 0)),
            pl.BlockSpec((BM, NC * NS), lambda i: (i, 0)),
        ],
        out_specs=[
            pl.BlockSpec((BM, D), lambda i: (i, 0)),
            pl.BlockSpec((BM, 1), lambda i: (i, 0)),
        ],
        out_shape=[
            jax.ShapeDtypeStruct((NP, D), jnp.float32),
            jax.ShapeDtypeStruct((NP, 1), jnp.float32),
        ],
    )(h, degt)


def _combine(acc2, hp, dis, b, W):
    def body(a_ref, hp_ref, dis_ref, b_ref, w_ref, o_ref):
        ssum = a_ref[0] + a_ref[1] - hp_ref[...]
        o = dis_ref[...] * ssum + b_ref[...]
        a = jnp.maximum(o, 0.0)
        o_ref[...] = (
            jax.lax.dot(a, w_ref[...], precision=jax.lax.Precision.HIGHEST)
            * dis_ref[...]
        )

    return pl.pallas_call(
        body,
        grid=(NP // BM,),
        in_specs=[
            pl.BlockSpec((NC, BM, D), lambda i: (0, i, 0)),
            pl.BlockSpec((BM, D), lambda i: (i, 0)),
            pl.BlockSpec((BM, 1), lambda i: (i, 0)),
            pl.BlockSpec((1, D), lambda i: (0, 0)),
            pl.BlockSpec((D, D), lambda i: (0, 0)),
        ],
        out_specs=pl.BlockSpec((BM, D), lambda i: (i, 0)),
        out_shape=jax.ShapeDtypeStruct((NP, D), jnp.float32),
    )(acc2, hp, dis, b, W)


def _final(acc2, hp, dis, b):
    def body(a_ref, hp_ref, dis_ref, b_ref, o_ref):
        ssum = a_ref[0] + a_ref[1] - hp_ref[...]
        o_ref[...] = dis_ref[...] * ssum + b_ref[...]

    return pl.pallas_call(
        body,
        grid=(NP // BM,),
        in_specs=[
            pl.BlockSpec((NC, BM, D), lambda i: (0, i, 0)),
            pl.BlockSpec((BM, D), lambda i: (i, 0)),
            pl.BlockSpec((BM, 1), lambda i: (i, 0)),
            pl.BlockSpec((1, D), lambda i: (0, 0)),
        ],
        out_specs=pl.BlockSpec((BM, D), lambda i: (i, 0)),
        out_shape=jax.ShapeDtypeStruct((NP, D), jnp.float32),
    )(acc2, hp, dis, b)


@jax.jit
def kernel(x, edge_index, W1, b1, W2, b2, W3, b3):
    src = edge_index[0].astype(jnp.int32)
    dst = edge_index[1].astype(jnp.int32)
    pad_e = jnp.full((E_PAD - E,), N, jnp.int32)
    src_p = jnp.concatenate([src, pad_e]).reshape(NC, NS, K, C)
    dst_p = jnp.concatenate([dst, pad_e]).reshape(NC, NS, K, C)
    x_p = jnp.zeros((NP, D), jnp.float32).at[:N].set(x)
    zeros1 = jnp.zeros((NP,), jnp.float32)

    degp = _sc_degree(dst_p, zeros1)
    h1 = _mm(x_p, W1)
    h1p, dis = _scale(h1, degp.T)
    acc1 = _sc_aggregate(h1p, src_p, dst_p)
    h2p = _combine(acc1, h1p, dis, b1.reshape(1, D), W2)
    acc2 = _sc_aggregate(h2p, src_p, dst_p)
    h3p = _combine(acc2, h2p, dis, b2.reshape(1, D), W3)
    acc3 = _sc_aggregate(h3p, src_p, dst_p)
    out = _final(acc3, h3p, dis, b3.reshape(1, D))
    return out[:N]


# speedup vs baseline: 1.3359x; 1.3359x over previous
import dataclasses
import functools

import jax
import jax.numpy as jnp
from jax import lax
from jax.experimental import pallas as pl
from jax.experimental.pallas import tpu as pltpu
from jax.experimental.pallas import tpu_sc as plsc

N = 10000
E = 320000
D = 128
NC, NS = 2, 16
C = 128
CH = D // NC
NP = 10240
RPT = NP // NS
K = 160
KH = K // NC
ET = K * C
E_PAD = ET * NS
NBUF = 5
BM = 256

_mesh = plsc.VectorSubcoreMesh(
    core_axis_name="c", subcore_axis_name="s", num_cores=NC, num_subcores=NS
)

_cp = dataclasses.replace(pltpu.CompilerParams(), needs_layout_passes=False)
_cp_lin = dataclasses.replace(pltpu.CompilerParams(), use_tc_tiling_on_sc=False)


@functools.partial(
    pl.kernel,
    out_type=jax.ShapeDtypeStruct((NC * NS, NP), jnp.float32),
    mesh=_mesh,
    compiler_params=_cp,
    scratch_types=[
        pltpu.VMEM((KH, C), jnp.int32),
        pltpu.VMEM((NP,), jnp.float32),
    ],
)
def _sc_degree(dstr_hbm, zeros_hbm, out_hbm, didx, cnt_v):
    c = lax.axis_index("c")
    sid = lax.axis_index("s")
    pltpu.sync_copy(zeros_hbm, cnt_v)
    pltpu.sync_copy(dstr_hbm.at[sid, pl.ds(c * KH, KH)], didx)
    ones = jnp.full((16,), 1.0, jnp.float32)

    @pl.loop(0, KH)
    def _(k):
        row = didx.at[k]

        @pl.loop(0, C // 16)
        def _(j):
            idx = row[pl.ds(j * 16, 16)]
            plsc.addupdate_scatter(cnt_v, [idx], ones)

    pltpu.sync_copy(cnt_v, out_hbm.at[c * NS + sid])


@functools.partial(
    pl.kernel,
    out_type=jax.ShapeDtypeStruct((NC, NP, CH), jnp.float32),
    mesh=_mesh,
    compiler_params=_cp_lin,
    scratch_types=[
        pltpu.VMEM((K, C), jnp.int32),
        pltpu.VMEM((K, C), jnp.int32),
        pltpu.VMEM((C, CH), jnp.float32),
        pltpu.VMEM((C, CH), jnp.float32),
        pltpu.VMEM((C, CH), jnp.float32),
        pltpu.VMEM((C, CH), jnp.float32),
        pltpu.VMEM((C, CH), jnp.float32),
        pltpu.VMEM_SHARED((NP, CH), jnp.float32),
        pltpu.SemaphoreType.DMA,
        pltpu.SemaphoreType.DMA,
        pltpu.SemaphoreType.DMA,
        pltpu.SemaphoreType.DMA,
        pltpu.SemaphoreType.DMA,
        pltpu.SemaphoreType.DMA,
        pltpu.SemaphoreType.DMA,
        pltpu.SemaphoreType.DMA,
        pltpu.SemaphoreType.DMA,
        pltpu.SemaphoreType.DMA,
    ],
)
def _sc_aggregate(hs_hbm, srcr_hbm, dstr_hbm, out_hbm, sidx, didx,
                  r0, r1, r2, r3, r4, acc_sh,
                  g0, g1, g2, g3, g4, s0, s1, s2, s3, s4):
    c = lax.axis_index("c")
    sid = lax.axis_index("s")
    rows = pl.ds(sid * RPT, RPT)
    htab = hs_hbm.at[c]
    pltpu.sync_copy(srcr_hbm.at[sid], sidx)
    pltpu.sync_copy(dstr_hbm.at[sid], didx)
    pltpu.sync_copy(htab.at[rows], acc_sh.at[rows])
    plsc.subcore_barrier()
    bufs = (r0, r1, r2, r3, r4)
    gsems = (g0, g1, g2, g3, g4)
    ssems = (s0, s1, s2, s3, s4)
    for b in range(NBUF):
        pltpu.async_copy(htab.at[sidx.at[b]], bufs[b], gsems[b])

    @pl.loop(0, K, step=NBUF)
    def _(k):
        for b in range(NBUF):
            kk = k + b
            pltpu.make_async_copy(htab.at[sidx.at[kk]], bufs[b], gsems[b]).wait()
            pltpu.async_copy(bufs[b], acc_sh.at[didx.at[kk]], ssems[b], add=True)
            pltpu.make_async_copy(bufs[b], acc_sh.at[didx.at[kk]], ssems[b]).wait()

            @pl.when(kk + NBUF < K)
            def _():
                pltpu.async_copy(htab.at[sidx.at[kk + NBUF]], bufs[b], gsems[b])

    plsc.subcore_barrier()
    pltpu.sync_copy(acc_sh.at[rows], out_hbm.at[c, rows])


def _mm(x, W):
    def body(x_ref, w_ref, o_ref):
        o_ref[...] = jax.lax.dot(
            x_ref[...], w_ref[...], precision=jax.lax.Precision.HIGHEST
        )

    return pl.pallas_call(
        body,
        grid=(NP // BM,),
        in_specs=[
            pl.BlockSpec((BM, D), lambda i: (i, 0)),
            pl.BlockSpec((D, D), lambda i: (0, 0)),
        ],
        out_specs=pl.BlockSpec((BM, D), lambda i: (i, 0)),
        out_shape=jax.ShapeDtypeStruct((NP, D), jnp.float32),
    )(x, W)


def _scale(h, degt):
    def body(h_ref, g_ref, hs_ref, dis_ref):
        deg = jnp.sum(g_ref[...], axis=1, keepdims=True) + 1.0
        dis = jax.lax.rsqrt(deg)
        hp = h_ref[...] * dis
        hs_ref[0, :, :] = hp[:, :CH]
        hs_ref[1, :, :] = hp[:, CH:]
        dis_ref[...] = dis

    return pl.pallas_call(
        body,
        grid=(NP // BM,),
        in_specs=[
            pl.BlockSpec((BM, D), lambda i: (i, 0)),
            pl.BlockSpec((BM, NC * NS), lambda i: (i, 0)),
        ],
        out_specs=[
            pl.BlockSpec((NC, BM, CH), lambda i: (0, i, 0)),
            pl.BlockSpec((BM, 1), lambda i: (i, 0)),
        ],
        out_shape=[
            jax.ShapeDtypeStruct((NC, NP, CH), jnp.float32),
            jax.ShapeDtypeStruct((NP, 1), jnp.float32),
        ],
    )(h, degt)


def _combine(acc2, dis, b, W):
    def body(a_ref, dis_ref, b_ref, w_ref, o_ref):
        ssum = jnp.concatenate([a_ref[0], a_ref[1]], axis=1)
        o = dis_ref[...] * ssum + b_ref[...]
        a = jnp.maximum(o, 0.0)
        hn = (
            jax.lax.dot(a, w_ref[...], precision=jax.lax.Precision.HIGHEST)
            * dis_ref[...]
        )
        o_ref[0, :, :] = hn[:, :CH]
        o_ref[1, :, :] = hn[:, CH:]

    return pl.pallas_call(
        body,
        grid=(NP // BM,),
        in_specs=[
            pl.BlockSpec((NC, BM, CH), lambda i: (0, i, 0)),
            pl.BlockSpec((BM, 1), lambda i: (i, 0)),
            pl.BlockSpec((1, D), lambda i: (0, 0)),
            pl.BlockSpec((D, D), lambda i: (0, 0)),
        ],
        out_specs=pl.BlockSpec((NC, BM, CH), lambda i: (0, i, 0)),
        out_shape=jax.ShapeDtypeStruct((NC, NP, CH), jnp.float32),
    )(acc2, dis, b, W)


def _final(acc2, dis, b):
    def body(a_ref, dis_ref, b_ref, o_ref):
        ssum = jnp.concatenate([a_ref[0], a_ref[1]], axis=1)
        o_ref[...] = dis_ref[...] * ssum + b_ref[...]

    return pl.pallas_call(
        body,
        grid=(NP // BM,),
        in_specs=[
            pl.BlockSpec((NC, BM, CH), lambda i: (0, i, 0)),
            pl.BlockSpec((BM, 1), lambda i: (i, 0)),
            pl.BlockSpec((1, D), lambda i: (0, 0)),
        ],
        out_specs=pl.BlockSpec((BM, D), lambda i: (i, 0)),
        out_shape=jax.ShapeDtypeStruct((NP, D), jnp.float32),
    )(acc2, dis, b)


@jax.jit
def kernel(x, edge_index, W1, b1, W2, b2, W3, b3):
    src = edge_index[0].astype(jnp.int32)
    dst = edge_index[1].astype(jnp.int32)
    pad_e = jnp.full((E_PAD - E,), N, jnp.int32)
    src_p = jnp.concatenate([src, pad_e]).reshape(NS, K, C)
    dst_p = jnp.concatenate([dst, pad_e]).reshape(NS, K, C)
    x_p = jnp.zeros((NP, D), jnp.float32).at[:N].set(x)
    zeros1 = jnp.zeros((NP,), jnp.float32)

    degp = _sc_degree(dst_p, zeros1)
    h1 = _mm(x_p, W1)
    h1s, dis = _scale(h1, degp.T)
    acc1 = _sc_aggregate(h1s, src_p, dst_p)
    h2s = _combine(acc1, dis, b1.reshape(1, D), W2)
    acc2 = _sc_aggregate(h2s, src_p, dst_p)
    h3s = _combine(acc2, dis, b2.reshape(1, D), W3)
    acc3 = _sc_aggregate(h3s, src_p, dst_p)
    out = _final(acc3, dis, b3.reshape(1, D))
    return out[:N]
